# Initial kernel scaffold; baseline (speedup 1.0000x reference)
import jax
import jax.numpy as jnp
from jax.experimental import pallas as pl


def kernel(x, W_enc, b_enc, W_dec, b_dec):
    raise NotImplementedError("write your pallas kernel here")


# speedup vs baseline: 5.9902x; 5.9902x over previous
import functools

import jax
import jax.numpy as jnp
from jax.experimental import pallas as pl

D_IN = 2048
D_OUT = 2048
N_FEAT = 16384
TOP_K = 32
N_TOK = 4096


PREP_TB = 1024


def _prep_body(x_ref, xp_ref, norm_ref):
    xb = x_ref[...]
    n = jnp.sqrt(jnp.sum(xb * xb, axis=1, keepdims=True))
    xp_ref[...] = xb / (n + 1e-8)
    norm_ref[...] = n


def _prep(x):
    nt = N_TOK // PREP_TB
    return pl.pallas_call(
        _prep_body,
        grid=(nt,),
        in_specs=[pl.BlockSpec((PREP_TB, D_IN), lambda t: (t, 0))],
        out_specs=[
            pl.BlockSpec((PREP_TB, D_IN), lambda t: (t, 0)),
            pl.BlockSpec((PREP_TB, 1), lambda t: (t, 0)),
        ],
        out_shape=[
            jax.ShapeDtypeStruct((N_TOK, D_IN), jnp.float32),
            jax.ShapeDtypeStruct((N_TOK, 1), jnp.float32),
        ],
    )(x)


ENC_TB = 512
ENC_FB = 1024


def _encode_body(xp_ref, w_ref, b_ref, pre_ref):
    acc = jax.lax.dot_general(
        xp_ref[...], w_ref[...], (((1,), (1,)), ((), ())),
        preferred_element_type=jnp.float32)
    pre_ref[...] = acc + b_ref[...]


def _encode(xp, W_enc, b_enc2d):
    nf = N_FEAT // ENC_FB
    nt = N_TOK // ENC_TB
    return pl.pallas_call(
        _encode_body,
        grid=(nf, nt),
        in_specs=[
            pl.BlockSpec((ENC_TB, D_IN), lambda f, t: (t, 0)),
            pl.BlockSpec((ENC_FB, D_IN), lambda f, t: (f, 0)),
            pl.BlockSpec((1, ENC_FB), lambda f, t: (0, f)),
        ],
        out_specs=pl.BlockSpec((ENC_TB, ENC_FB), lambda f, t: (t, f)),
        out_shape=jax.ShapeDtypeStruct((N_TOK, N_FEAT), jnp.float32),
    )(xp, W_enc, b_enc2d)


SEL_TB = 128


def _select_body(pre_ref, hid_ref):
    a = pre_ref[...]
    u = jax.lax.bitcast_convert_type(a, jnp.int32)
    kk = jnp.where(u < 0, u ^ jnp.int32(0x7FFFFFFF), u)

    def cnt_ge(t):
        return jnp.sum((kk >= t).astype(jnp.int32), axis=1, keepdims=True)

    int_min = jnp.int32(-2147483648)
    t0 = jnp.where(cnt_ge(jnp.zeros((SEL_TB, 1), jnp.int32)) >= TOP_K,
                   jnp.int32(0), int_min)

    def step(i, t):
        bit = jax.lax.shift_left(jnp.int32(1), jnp.int32(30) - i)
        cand = t | bit
        return jnp.where(cnt_ge(cand) >= TOP_K, cand, t)

    t = jax.lax.fori_loop(0, 31, step, t0)
    mask = kk >= t
    hid_ref[...] = jnp.where(mask, jnp.maximum(a, 0.0), 0.0)


def _select(pre_acts):
    nt = N_TOK // SEL_TB
    return pl.pallas_call(
        _select_body,
        grid=(nt,),
        in_specs=[pl.BlockSpec((SEL_TB, N_FEAT), lambda t: (t, 0))],
        out_specs=pl.BlockSpec((SEL_TB, N_FEAT), lambda t: (t, 0)),
        out_shape=jax.ShapeDtypeStruct((N_TOK, N_FEAT), jnp.float32),
    )(pre_acts)


DEC_TB = 1024
DEC_KB = 1024


def _decode_body(hid_ref, w_ref, b_ref, norm_ref, out_ref, *, nk):
    k = pl.program_id(1)

    @pl.when(k == 0)
    def _():
        out_ref[...] = jnp.zeros_like(out_ref)

    out_ref[...] += jax.lax.dot_general(
        hid_ref[...], w_ref[...], (((1,), (1,)), ((), ())),
        preferred_element_type=jnp.float32)

    @pl.when(k == nk - 1)
    def _():
        out_ref[...] = (out_ref[...] + b_ref[...]) * norm_ref[...]


def _decode(hidden, W_dec, b_dec2d, x_norm):
    nt = N_TOK // DEC_TB
    nk = N_FEAT // DEC_KB
    return pl.pallas_call(
        functools.partial(_decode_body, nk=nk),
        grid=(nt, nk),
        in_specs=[
            pl.BlockSpec((DEC_TB, DEC_KB), lambda t, k: (t, k)),
            pl.BlockSpec((D_OUT, DEC_KB), lambda t, k: (0, k)),
            pl.BlockSpec((1, D_OUT), lambda t, k: (0, 0)),
            pl.BlockSpec((DEC_TB, 1), lambda t, k: (t, 0)),
        ],
        out_specs=pl.BlockSpec((DEC_TB, D_OUT), lambda t, k: (t, 0)),
        out_shape=jax.ShapeDtypeStruct((N_TOK, D_OUT), jnp.float32),
    )(hidden, W_dec, b_dec2d, x_norm)


def kernel(x, W_enc, b_enc, W_dec, b_dec):
    x_proc, x_norm = _prep(x)
    pre_acts = _encode(x_proc, W_enc, b_enc.reshape(1, N_FEAT))
    hidden = _select(pre_acts)
    out = _decode(hidden, W_dec, b_dec.reshape(1, D_OUT), x_norm)
    auxiliary_loss = jnp.zeros((), jnp.float32)
    return (out, hidden, pre_acts, auxiliary_loss)
